# baseline (speedup 1.0000x reference)
import functools

import jax
import jax.numpy as jnp
from jax.experimental import pallas as pl
from jax.experimental.pallas import tpu as pltpu

_TOP_K = 2
_BLOCK_T = 1024


def _router_block(x_ref, wt_ref, idx_ref, w_ref):
    x = x_ref[...]
    wt = wt_ref[...]
    logits = jnp.dot(x, wt, preferred_element_type=jnp.float32)
    m = jnp.max(logits, axis=-1, keepdims=True)
    e = jnp.exp(logits - m)
    s = e / jnp.sum(e, axis=-1, keepdims=True)
    n_e = s.shape[-1]
    lane = jax.lax.broadcasted_iota(jnp.int32, s.shape, 1)
    m1 = jnp.max(s, axis=-1, keepdims=True)
    i1 = jnp.min(jnp.where(s == m1, lane, n_e), axis=-1, keepdims=True)
    s2 = jnp.where(lane == i1, -1.0, s)
    m2 = jnp.max(s2, axis=-1, keepdims=True)
    i2 = jnp.min(jnp.where(s2 == m2, lane, n_e), axis=-1, keepdims=True)
    denom = m1 + m2 + 1e-20
    idx_ref[...] = jnp.concatenate([i1, i2], axis=-1)
    w_ref[...] = jnp.concatenate([m1 / denom, m2 / denom], axis=-1)


@jax.jit
def kernel(hidden_states, weight):
    bsz, seq_len, h = hidden_states.shape
    n_tok = bsz * seq_len
    n_exp = weight.shape[0]
    x = hidden_states.reshape(n_tok, h).astype(jnp.float32)
    wt = weight.astype(jnp.float32).T

    grid = (n_tok // _BLOCK_T,)
    topk_idx, topk_w = pl.pallas_call(
        _router_block,
        grid=grid,
        in_specs=[
            pl.BlockSpec((_BLOCK_T, h), lambda i: (i, 0)),
            pl.BlockSpec((h, n_exp), lambda i: (0, 0)),
        ],
        out_specs=[
            pl.BlockSpec((_BLOCK_T, _TOP_K), lambda i: (i, 0)),
            pl.BlockSpec((_BLOCK_T, _TOP_K), lambda i: (i, 0)),
        ],
        out_shape=[
            jax.ShapeDtypeStruct((n_tok, _TOP_K), jnp.int32),
            jax.ShapeDtypeStruct((n_tok, _TOP_K), jnp.float32),
        ],
        compiler_params=pltpu.CompilerParams(
            dimension_semantics=("arbitrary",),
        ),
    )(x, wt)
    return (topk_idx, topk_w)
